# Initial kernel scaffold; baseline (speedup 1.0000x reference)
import functools

import jax
import jax.numpy as jnp
from jax import lax
from jax.experimental import pallas as pl
from jax.experimental.pallas import tpu as pltpu
from jax.experimental.pallas import tpu_sc as plsc

_N = 10000
_D = 128
_E = 320000
_NCLS = 64

_NC = 2
_NS = 16
_NW = _NC * _NS
_CHUNK = 128
_C = 80
_EPW = _C * _CHUNK
_EPAD = _NW * _EPW
_NACC = 10112
_RPT = _NACC // _NS


def _sc_pass(h, src_w, dst_w, *, with_deg):
    mesh = plsc.VectorSubcoreMesh(core_axis_name="c", subcore_axis_name="s")
    out_type = [jax.ShapeDtypeStruct((_NC, _NACC, _D), jnp.float32)]
    scratch = [
        pltpu.VMEM((_C, _CHUNK), jnp.int32),
        pltpu.VMEM((_C, _CHUNK), jnp.int32),
        pltpu.VMEM((_CHUNK, _D), jnp.float32),
        pltpu.VMEM((_CHUNK, _D), jnp.float32),
        pltpu.VMEM((8, _D), jnp.float32),
        pltpu.VMEM_SHARED((_NACC, _D), jnp.float32),
        pltpu.SemaphoreType.DMA,
        pltpu.SemaphoreType.DMA,
    ]
    if with_deg:
        out_type.append(jax.ShapeDtypeStruct((_NC, _NACC, 16), jnp.float32))
        scratch += [
            pltpu.VMEM((_CHUNK, 16), jnp.float32),
            pltpu.VMEM((8, 16), jnp.float32),
            pltpu.VMEM_SHARED((_NACC, 16), jnp.float32),
        ]

    def body(h_hbm, src_hbm, dst_hbm, *refs):
        if with_deg:
            (agg_hbm, deg_hbm, src_v, dst_v, rows0, rows1, zrow, acc,
             sem0, sem1, ones_v, zrow16, dacc) = refs
        else:
            (agg_hbm, src_v, dst_v, rows0, rows1, zrow, acc,
             sem0, sem1) = refs
        cid = lax.axis_index("c")
        sid = lax.axis_index("s")
        wid = sid * _NC + cid
        base = sid * _RPT

        pltpu.sync_copy(src_hbm.at[wid], src_v)
        pltpu.sync_copy(dst_hbm.at[wid], dst_v)

        @pl.loop(0, 8)
        def _(r):
            @pl.loop(0, _D // 16)
            def _(c):
                zrow[r, pl.ds(c * 16, 16)] = jnp.zeros((16,), jnp.float32)

        @pl.loop(0, _RPT // 8)
        def _(k):
            pltpu.sync_copy(zrow, acc.at[pl.ds(base + k * 8, 8)])

        if with_deg:
            @pl.loop(0, _CHUNK)
            def _(r):
                ones_v[r, :] = jnp.ones((16,), jnp.float32)

            @pl.loop(0, 8)
            def _(r):
                zrow16[r, :] = jnp.zeros((16,), jnp.float32)

            @pl.loop(0, _RPT // 8)
            def _(k):
                pltpu.sync_copy(zrow16, dacc.at[pl.ds(base + k * 8, 8)])

        plsc.subcore_barrier()

        pltpu.async_copy(h_hbm.at[src_v.at[0]], rows0, sem0)
        pltpu.async_copy(h_hbm.at[src_v.at[1]], rows1, sem1)

        @pl.loop(0, _C, step=2)
        def _(j):
            pltpu.make_async_copy(h_hbm.at[src_v.at[j]], rows0, sem0).wait()
            pltpu.sync_copy(rows0, acc.at[dst_v.at[j]], add=True)
            if with_deg:
                pltpu.sync_copy(ones_v, dacc.at[dst_v.at[j]], add=True)

            @pl.when(j + 2 < _C)
            def _():
                pltpu.async_copy(h_hbm.at[src_v.at[j + 2]], rows0, sem0)

            pltpu.make_async_copy(h_hbm.at[src_v.at[j + 1]], rows1, sem1).wait()
            pltpu.sync_copy(rows1, acc.at[dst_v.at[j + 1]], add=True)
            if with_deg:
                pltpu.sync_copy(ones_v, dacc.at[dst_v.at[j + 1]], add=True)

            @pl.when(j + 3 < _C)
            def _():
                pltpu.async_copy(h_hbm.at[src_v.at[j + 3]], rows1, sem1)

        plsc.subcore_barrier()
        pltpu.sync_copy(acc.at[pl.ds(base, _RPT)],
                        agg_hbm.at[cid].at[pl.ds(base, _RPT)])
        if with_deg:
            pltpu.sync_copy(dacc.at[pl.ds(base, _RPT)],
                            deg_hbm.at[cid].at[pl.ds(base, _RPT)])

    f = pl.kernel(body, out_type=tuple(out_type), mesh=mesh,
                  scratch_types=scratch)
    return f(h, src_w, dst_w)


_BLK = 1000


def _tc_layer(h, agg_p, deg_p, w_self, w_neigh, b, fc_w=None, fc_b=None):
    n_out = _NCLS if fc_w is not None else _D
    in_specs = [
        pl.BlockSpec((_BLK, _D), lambda i: (i, 0)),
        pl.BlockSpec((_NC, _BLK, _D), lambda i: (0, i, 0)),
        pl.BlockSpec((_NC, _BLK, 16), lambda i: (0, i, 0)),
        pl.BlockSpec((_D, _D), lambda i: (0, 0)),
        pl.BlockSpec((_D, _D), lambda i: (0, 0)),
        pl.BlockSpec((1, _D), lambda i: (0, 0)),
    ]
    args = [h, agg_p, deg_p, w_self, w_neigh, b.reshape(1, _D)]
    if fc_w is not None:
        in_specs += [pl.BlockSpec((_D, _NCLS), lambda i: (0, 0)),
                     pl.BlockSpec((1, _NCLS), lambda i: (0, 0))]
        args += [fc_w, fc_b.reshape(1, _NCLS)]

    def body(h_ref, p_ref, d_ref, ws_ref, wn_ref, b_ref, *rest):
        if fc_w is not None:
            fw_ref, fb_ref, o_ref = rest
        else:
            (o_ref,) = rest
        agg = p_ref[0] + p_ref[1]
        deg = d_ref[0, :, 0:1] + d_ref[1, :, 0:1]
        hn = agg / jnp.maximum(deg, 1.0)
        y = jnp.dot(h_ref[...], ws_ref[...], preferred_element_type=jnp.float32)
        y = y + jnp.dot(hn, wn_ref[...], preferred_element_type=jnp.float32)
        y = y + b_ref[...]
        if fc_w is not None:
            y = jnp.dot(y, fw_ref[...], preferred_element_type=jnp.float32)
            y = y + fb_ref[...]
        o_ref[...] = y

    return pl.pallas_call(
        body,
        grid=(_N // _BLK,),
        in_specs=in_specs,
        out_specs=pl.BlockSpec((_BLK, n_out), lambda i: (i, 0)),
        out_shape=jax.ShapeDtypeStruct((_N, n_out), jnp.float32),
    )(*args)


def kernel(x, edge_index, W_self_0, W_neigh_0, b_0, W_self_1, W_neigh_1, b_1,
           W_self_2, W_neigh_2, b_2, fc1_W, fc1_b):
    src = edge_index[0]
    dst = edge_index[1]
    pad = _EPAD - _E
    src_w = jnp.concatenate(
        [src, jnp.zeros((pad,), jnp.int32)]).reshape(_NW, _C, _CHUNK)
    dst_w = jnp.concatenate(
        [dst, jnp.full((pad,), _N, jnp.int32)]).reshape(_NW, _C, _CHUNK)

    p0, degp = _sc_pass(x, src_w, dst_w, with_deg=True)
    h1 = _tc_layer(x, p0, degp, W_self_0, W_neigh_0, b_0)
    (p1,) = _sc_pass(h1, src_w, dst_w, with_deg=False)
    h2 = _tc_layer(h1, p1, degp, W_self_1, W_neigh_1, b_1)
    (p2,) = _sc_pass(h2, src_w, dst_w, with_deg=False)
    out = _tc_layer(h2, p2, degp, W_self_2, W_neigh_2, b_2, fc1_W, fc1_b)
    return out


# speedup vs baseline: 3.2718x; 3.2718x over previous
import jax
import jax.numpy as jnp
from jax import lax
from jax.experimental import pallas as pl
from jax.experimental.pallas import tpu as pltpu
from jax.experimental.pallas import tpu_sc as plsc

_N = 10000
_D = 128
_E = 320000
_NCLS = 64

_NC = 2
_NS = 16
_NW = _NC * _NS
_CHUNK = 128
_C = 80
_HALF = _C // 2
_EPW = _C * _CHUNK
_EPAD = _NW * _EPW
_NACC = 10112
_RPT = _NACC // _NS

_MESH = plsc.VectorSubcoreMesh(core_axis_name="c", subcore_axis_name="s")


def _sc_pass(h, src_w, dst_w):
    def body(h_hbm, src_hbm, dst_hbm, agg_hbm, src_v, dst_v, rows0, rows1,
             acc, sem0, sem1):
        cid = lax.axis_index("c")
        sid = lax.axis_index("s")
        wid = sid * _NC + cid
        base = sid * _RPT

        @pl.loop(0, _CHUNK)
        def _(r):
            @pl.loop(0, _D // 16)
            def _(c):
                rows0[r, pl.ds(c * 16, 16)] = jnp.zeros((16,), jnp.float32)

        @pl.loop(0, 4)
        def _(k):
            pltpu.sync_copy(rows0, acc.at[pl.ds(base + k * _CHUNK, _CHUNK)])

        pltpu.sync_copy(rows0.at[pl.ds(0, _RPT - 4 * _CHUNK)],
                        acc.at[pl.ds(base + 4 * _CHUNK, _RPT - 4 * _CHUNK)])
        plsc.subcore_barrier()

        for half in range(2):
            off = half * _HALF
            pltpu.sync_copy(src_hbm.at[wid].at[pl.ds(off, _HALF)], src_v)
            pltpu.sync_copy(dst_hbm.at[wid].at[pl.ds(off, _HALF)], dst_v)
            pltpu.async_copy(h_hbm.at[src_v.at[0]], rows0, sem0)
            pltpu.async_copy(h_hbm.at[src_v.at[1]], rows1, sem1)

            @pl.loop(0, _HALF, step=2)
            def _(j):
                pltpu.make_async_copy(h_hbm.at[src_v.at[j]], rows0, sem0).wait()
                pltpu.sync_copy(rows0, acc.at[dst_v.at[j]], add=True)

                @pl.when(j + 2 < _HALF)
                def _():
                    pltpu.async_copy(h_hbm.at[src_v.at[j + 2]], rows0, sem0)

                pltpu.make_async_copy(h_hbm.at[src_v.at[j + 1]], rows1,
                                      sem1).wait()
                pltpu.sync_copy(rows1, acc.at[dst_v.at[j + 1]], add=True)

                @pl.when(j + 3 < _HALF)
                def _():
                    pltpu.async_copy(h_hbm.at[src_v.at[j + 3]], rows1, sem1)

        plsc.subcore_barrier()
        pltpu.sync_copy(acc.at[pl.ds(base, _RPT)],
                        agg_hbm.at[cid].at[pl.ds(base, _RPT)])

    f = pl.kernel(
        body,
        out_type=jax.ShapeDtypeStruct((_NC, _NACC, _D), jnp.float32),
        mesh=_MESH,
        scratch_types=[
            pltpu.VMEM((_HALF, _CHUNK), jnp.int32),
            pltpu.VMEM((_HALF, _CHUNK), jnp.int32),
            pltpu.VMEM((_CHUNK, _D), jnp.float32),
            pltpu.VMEM((_CHUNK, _D), jnp.float32),
            pltpu.VMEM_SHARED((_NACC, _D), jnp.float32),
            pltpu.SemaphoreType.DMA,
            pltpu.SemaphoreType.DMA,
        ],
    )
    return f(h, src_w, dst_w)


def _sc_deg(dst_w):
    def body(dst_hbm, deg_hbm, dst_v, ones_v, dacc, sem):
        del sem
        cid = lax.axis_index("c")
        sid = lax.axis_index("s")
        wid = sid * _NC + cid
        base = sid * _RPT

        pltpu.sync_copy(dst_hbm.at[wid], dst_v)

        @pl.loop(0, _CHUNK)
        def _(r):
            @pl.loop(0, _D // 16)
            def _(c):
                ones_v[r, pl.ds(c * 16, 16)] = jnp.zeros((16,), jnp.float32)

        @pl.loop(0, _RPT // 8)
        def _(k):
            pltpu.sync_copy(ones_v.at[pl.ds(0, 8)],
                            dacc.at[pl.ds(base + k * 8, 8)])

        @pl.loop(0, _CHUNK)
        def _(r):
            @pl.loop(0, _D // 16)
            def _(c):
                ones_v[r, pl.ds(c * 16, 16)] = jnp.ones((16,), jnp.float32)

        plsc.subcore_barrier()

        @pl.loop(0, _C)
        def _(j):
            pltpu.sync_copy(ones_v, dacc.at[dst_v.at[j]], add=True)

        plsc.subcore_barrier()
        pltpu.sync_copy(dacc.at[pl.ds(base, _RPT)],
                        deg_hbm.at[cid].at[pl.ds(base, _RPT)])

    f = pl.kernel(
        body,
        out_type=jax.ShapeDtypeStruct((_NC, _NACC, _D), jnp.float32),
        mesh=_MESH,
        scratch_types=[
            pltpu.VMEM((_C, _CHUNK), jnp.int32),
            pltpu.VMEM((_CHUNK, _D), jnp.float32),
            pltpu.VMEM_SHARED((_NACC, _D), jnp.float32),
            pltpu.SemaphoreType.DMA,
        ],
    )
    return f(dst_w)


_BLK = 1000


def _tc_layer(h, agg_p, deg_p, w_self, w_neigh, b, fc_w=None, fc_b=None):
    n_out = _NCLS if fc_w is not None else _D
    in_specs = [
        pl.BlockSpec((_BLK, _D), lambda i: (i, 0)),
        pl.BlockSpec((_NC, _BLK, _D), lambda i: (0, i, 0)),
        pl.BlockSpec((_NC, _BLK, _D), lambda i: (0, i, 0)),
        pl.BlockSpec((_D, _D), lambda i: (0, 0)),
        pl.BlockSpec((_D, _D), lambda i: (0, 0)),
        pl.BlockSpec((1, _D), lambda i: (0, 0)),
    ]
    args = [h, agg_p, deg_p, w_self, w_neigh, b.reshape(1, _D)]
    if fc_w is not None:
        in_specs += [pl.BlockSpec((_D, _NCLS), lambda i: (0, 0)),
                     pl.BlockSpec((1, _NCLS), lambda i: (0, 0))]
        args += [fc_w, fc_b.reshape(1, _NCLS)]

    def body(h_ref, p_ref, d_ref, ws_ref, wn_ref, b_ref, *rest):
        if fc_w is not None:
            fw_ref, fb_ref, o_ref = rest
        else:
            (o_ref,) = rest
        agg = p_ref[0] + p_ref[1]
        deg = d_ref[0, :, 0:1] + d_ref[1, :, 0:1]
        hn = agg / jnp.maximum(deg, 1.0)
        y = jnp.dot(h_ref[...], ws_ref[...], preferred_element_type=jnp.float32)
        y = y + jnp.dot(hn, wn_ref[...], preferred_element_type=jnp.float32)
        y = y + b_ref[...]
        if fc_w is not None:
            y = jnp.dot(y, fw_ref[...], preferred_element_type=jnp.float32)
            y = y + fb_ref[...]
        o_ref[...] = y

    return pl.pallas_call(
        body,
        grid=(_N // _BLK,),
        in_specs=in_specs,
        out_specs=pl.BlockSpec((_BLK, n_out), lambda i: (i, 0)),
        out_shape=jax.ShapeDtypeStruct((_N, n_out), jnp.float32),
    )(*args)


def kernel(x, edge_index, W_self_0, W_neigh_0, b_0, W_self_1, W_neigh_1, b_1,
           W_self_2, W_neigh_2, b_2, fc1_W, fc1_b):
    src = edge_index[0]
    dst = edge_index[1]
    pad = _EPAD - _E
    src_w = jnp.concatenate(
        [src, jnp.zeros((pad,), jnp.int32)]).reshape(_NW, _C, _CHUNK)
    dst_w = jnp.concatenate(
        [dst, jnp.full((pad,), _N, jnp.int32)]).reshape(_NW, _C, _CHUNK)

    degp = _sc_deg(dst_w)
    p0 = _sc_pass(x, src_w, dst_w)
    h1 = _tc_layer(x, p0, degp, W_self_0, W_neigh_0, b_0)
    p1 = _sc_pass(h1, src_w, dst_w)
    h2 = _tc_layer(h1, p1, degp, W_self_1, W_neigh_1, b_1)
    p2 = _sc_pass(h2, src_w, dst_w)
    out = _tc_layer(h2, p2, degp, W_self_2, W_neigh_2, b_2, fc1_W, fc1_b)
    return out
